# baseline (speedup 1.0000x reference)
import dataclasses
import functools

import jax
import jax.numpy as jnp
from jax import lax
from jax.experimental import pallas as pl
from jax.experimental.pallas import tpu as pltpu
from jax.experimental.pallas import tpu_sc as plsc

_AH = 7
_AW = 7
_PTS = _AH * _AW
_N = 1000
_C = 256
_SIDES = (128, 64, 32, 16)
_SCALES = (0.25, 0.125, 0.0625, 0.03125)
_REF_AREAS = (1024.0, 4096.0, 16384.0, 65536.0)
_LVL_OFF = (0, 32768, 40960, 43008)
_ROWS = 43520
_NQ = _N * _PTS
_NP = 49152
_PREP_R = 384
_PREP_C = 128

_NW = 32
_PER_W = _NP // _NW
_G = 48
_NCH = _PER_W // _G


def _prep_body(x1r, y1r, x2r, y2r, br,
               i0, i1, i2, i3, w0, w1, w2, w3):
    x1 = x1r[...]
    y1 = y1r[...]
    x2 = x2r[...]
    y2 = y2r[...]
    b = br[...]
    rows = lax.broadcasted_iota(jnp.int32, (_PREP_R, _PREP_C), 0)
    cols = lax.broadcasted_iota(jnp.int32, (_PREP_R, _PREP_C), 1)
    q = rows * _PREP_C + cols
    p = q % _PTS
    py = p // _AW
    px = p % _AW

    area = (x2 - x1 + 1.0) * (y2 - y1 + 1.0)
    lvl = jnp.zeros_like(q)
    best = jnp.abs(jnp.log(jnp.sqrt(area / _REF_AREAS[0])) / 0.6931472)
    for l in range(1, 4):
        v = jnp.abs(jnp.log(jnp.sqrt(area / _REF_AREAS[l])) / 0.6931472)
        upd = v < best
        lvl = jnp.where(upd, l, lvl)
        best = jnp.where(upd, v, best)

    def sel_f(vals):
        return jnp.where(lvl == 0, vals[0],
                         jnp.where(lvl == 1, vals[1],
                                   jnp.where(lvl == 2, vals[2], vals[3])))

    scale = sel_f([jnp.float32(s) for s in _SCALES])
    side_f = sel_f([jnp.float32(s) for s in _SIDES])
    side_i = sel_f([jnp.int32(s) for s in _SIDES])
    off = sel_f([jnp.int32(s) for s in _LVL_OFF])
    hw = side_i * side_i

    x1s = x1 * scale
    y1s = y1 * scale
    x2s = x2 * scale
    y2s = y2 * scale
    roi_w = jnp.maximum(x2s - x1s, 1.0)
    roi_h = jnp.maximum(y2s - y1s, 1.0)
    bin_w = roi_w / _AW
    bin_h = roi_h / _AH
    sx = x1s + bin_w * (px.astype(jnp.float32) + 0.5)
    sy = y1s + bin_h * (py.astype(jnp.float32) + 0.5)
    valid = (sy > -1.0) & (sy < side_f) & (sx > -1.0) & (sx < side_f)
    yc = jnp.clip(sy, 0.0, side_f - 1.0)
    xc = jnp.clip(sx, 0.0, side_f - 1.0)
    y0f = jnp.floor(yc)
    x0f = jnp.floor(xc)
    y0 = y0f.astype(jnp.int32)
    x0 = x0f.astype(jnp.int32)
    y1i = jnp.minimum(y0 + 1, side_i - 1)
    x1i = jnp.minimum(x0 + 1, side_i - 1)
    ly = yc - y0f
    lx = xc - x0f
    hy = 1.0 - ly
    hx = 1.0 - lx
    vm = (valid & (q < _NQ)).astype(jnp.float32)

    w0[...] = hy * hx * vm
    w1[...] = hy * lx * vm
    w2[...] = ly * hx * vm
    w3[...] = ly * lx * vm
    rb = off + b * hw
    i0[...] = rb + y0 * side_i + x0
    i1[...] = rb + y0 * side_i + x1i
    i2[...] = rb + y1i * side_i + x0
    i3[...] = rb + y1i * side_i + x1i


def _prep_call(x1r, y1r, x2r, y2r, br):
    i32 = jax.ShapeDtypeStruct((_PREP_R, _PREP_C), jnp.int32)
    f32 = jax.ShapeDtypeStruct((_PREP_R, _PREP_C), jnp.float32)
    return pl.pallas_call(
        _prep_body,
        out_shape=[i32, i32, i32, i32, f32, f32, f32, f32],
    )(x1r, y1r, x2r, y2r, br)


def _sc_body(tab, i0, i1, i2, i3, w0, w1, w2, w3, out,
             iv0, iv1, iv2, iv3, wv0, wv1, wv2, wv3,
             ra0, ra1, ra2, ra3, rb0, rb1, rb2, rb3, acc,
             gs0, gs1, ss):
    wid = lax.axis_index("s") * 2 + lax.axis_index("c")
    base = wid * _PER_W
    sl = pl.ds(base, _PER_W)
    cps = [
        pltpu.async_copy(i0.at[sl], iv0, gs0),
        pltpu.async_copy(i1.at[sl], iv1, gs0),
        pltpu.async_copy(i2.at[sl], iv2, gs0),
        pltpu.async_copy(i3.at[sl], iv3, gs0),
        pltpu.async_copy(w0.at[sl], wv0, gs0),
        pltpu.async_copy(w1.at[sl], wv1, gs0),
        pltpu.async_copy(w2.at[sl], wv2, gs0),
        pltpu.async_copy(w3.at[sl], wv3, gs0),
    ]
    for cp in cps:
        cp.wait()

    ivs = (iv0, iv1, iv2, iv3)
    rsets = ((ra0, ra1, ra2, ra3), (rb0, rb1, rb2, rb3))
    gsems = (gs0, gs1)

    def gathers(cc, k):
        off = cc * _G
        return [pltpu.make_async_copy(tab.at[iv.at[pl.ds(off, _G)]], r,
                                      gsems[k])
                for iv, r in zip(ivs, rsets[k])]

    def store(cc):
        return pltpu.make_async_copy(acc, out.at[pl.ds(base + cc * _G, _G)],
                                     ss)

    for g in gathers(0, 0):
        g.start()
    for g in gathers(1, 1):
        g.start()

    @pl.loop(0, _NCH, step=2)
    def _pair(c):
        for k in (0, 1):
            cc = c + k
            rs = rsets[k]
            for g in gathers(cc, k):
                g.wait()

            @pl.when(cc > 0)
            def _():
                store(cc).wait()

            off = cc * _G

            @plsc.parallel_loop(0, _G, unroll=2)
            def _pt(i):
                qi = jnp.full((16,), off + i, dtype=jnp.int32)
                a0 = plsc.load_gather(wv0, [qi])
                a1 = plsc.load_gather(wv1, [qi])
                a2 = plsc.load_gather(wv2, [qi])
                a3 = plsc.load_gather(wv3, [qi])
                for j in range(_C // 16):
                    cs = pl.ds(16 * j, 16)
                    acc[i, cs] = ((a0 * rs[0][i, cs] + a1 * rs[1][i, cs])
                                  + a2 * rs[2][i, cs] + a3 * rs[3][i, cs])

            store(cc).start()

            @pl.when(cc + 2 < _NCH)
            def _():
                for g in gathers(cc + 2, k):
                    g.start()

    store(_NCH - 1).wait()


def _sc_call(table, i0, i1, i2, i3, w0, w1, w2, w3):
    cp = pltpu.CompilerParams()
    if "needs_layout_passes" in pltpu.CompilerParams.__dataclass_fields__:
        cp = dataclasses.replace(cp, needs_layout_passes=False)
    mesh = plsc.VectorSubcoreMesh(core_axis_name="c", subcore_axis_name="s")
    run = functools.partial(
        pl.kernel,
        out_type=jax.ShapeDtypeStruct((_NP, _C), jnp.float32),
        mesh=mesh,
        compiler_params=cp,
        scratch_types=(
            [pltpu.VMEM((_PER_W,), jnp.int32)] * 4
            + [pltpu.VMEM((_PER_W,), jnp.float32)] * 4
            + [pltpu.VMEM((_G, _C), jnp.float32)] * 9
            + [pltpu.SemaphoreType.DMA] * 3
        ),
    )(_sc_body)
    return run(table, i0, i1, i2, i3, w0, w1, w2, w3)


def _expand(col):
    e = jnp.repeat(col, _PTS)
    e = jnp.concatenate([e, jnp.zeros((_NP - _NQ,), e.dtype)])
    return e.reshape(_PREP_R, _PREP_C)


def kernel(feat0, feat1, feat2, feat3, bboxes, batch_inds):
    feats = (feat0, feat1, feat2, feat3)
    table = jnp.concatenate(
        [jnp.transpose(f, (0, 2, 3, 1)).reshape(-1, _C) for f in feats], axis=0)

    bi = batch_inds.astype(jnp.int32)
    x1r = _expand(bboxes[:, 0])
    y1r = _expand(bboxes[:, 1])
    x2r = _expand(bboxes[:, 2])
    y2r = _expand(bboxes[:, 3])
    br = _expand(bi)

    i0, i1, i2, i3, w0, w1, w2, w3 = _prep_call(x1r, y1r, x2r, y2r, br)
    flat = lambda a: a.reshape(_NP)
    rows = _sc_call(table, flat(i0), flat(i1), flat(i2), flat(i3),
                    flat(w0), flat(w1), flat(w2), flat(w3))
    out = rows[:_NQ].reshape(_N, _PTS, _C).transpose(0, 2, 1)
    return out.reshape(_N, _C, _AH, _AW)
